# baseline (speedup 1.0000x reference)
import functools

import jax
import jax.numpy as jnp
from jax import lax
from jax.experimental import pallas as pl
from jax.experimental.pallas import tpu as pltpu
from jax.experimental.pallas import tpu_sc as plsc

N = 1_000_000
K = 50
NT = 16
PER = 62_496
SUB = 10_416
NSUB = PER // SUB
VECS = SUB // 16
UNR = 31
GRP = 31
NBUF = 4
TAIL_BASE = NT * PER
TAILV = (N - TAIL_BASE) // 16
HIST = 4096
H3 = 256
LSTRIDE = HIST + 8
HB = LSTRIDE * 16


def _key(v):
    b = plsc.bitcast(v, jnp.int32)
    return b ^ ((b >> 31) & jnp.int32(0x7FFFFFFF))


def _body(x_hbm, o_hbm, b0, b1_, b2, b3, tailbuf, hist16, histf, rh, colbuf,
          tmp, tmp2, tot2d, hist_sh, sum_sh, ctrl_sh,
          is0, is1, is2, is3, os0, os1, os2, os3):
    bufs = [b0, b1_, b2, b3]
    insems = [is0, is1, is2, is3]
    outsems = [os0, os1, os2, os3]
    sid = lax.axis_index("s")
    base = sid * PER
    lanes = lax.iota(jnp.int32, 16)
    laneoff = lanes * LSTRIDE
    ones = jnp.ones((16,), jnp.int32)
    zeros16 = jnp.zeros((16,), jnp.int32)
    is_tail = sid == NT - 1

    def clear_hist(h):
        def f(i, c):
            for l in range(16):
                hist16[pl.ds(l * LSTRIDE + i * 16, 16)] = zeros16
            return c
        lax.fori_loop(0, h // 16, f, 0)

    def in_copy(s, p):
        return pltpu.async_copy(
            x_hbm.at[pl.ds(base + s * SUB, SUB)], bufs[p], insems[p])

    def scan_round(bucket_fn):
        def group(ref, offs):
            vs = [ref[pl.ds(o, 16)] for o in offs]
            bs = [plsc.bitcast(v, jnp.int32) for v in vs]
            sg = [b >> 31 for b in bs]
            sg = [s & jnp.int32(0x7FFFFFFF) for s in sg]
            ks = [b ^ s for b, s in zip(bs, sg)]
            idxs, ms = bucket_fn(ks)
            if ms is None:
                for idx in idxs:
                    plsc.addupdate_scatter(hist16, [idx], ones)
            else:
                for idx, m in zip(idxs, ms):
                    plsc.addupdate_scatter(hist16, [idx], ones, mask=m)

        def one_vec(ref, off):
            group(ref, [off])

        def compute(ref):
            def f(i, c):
                for g in range(UNR // GRP):
                    b = (i * UNR + g * GRP) * 16
                    group(ref, [b + u * 16 for u in range(GRP)])
                return c
            lax.fori_loop(0, VECS // UNR, f, 0)

        hin = [None] * NBUF
        hin[0] = in_copy(0, 0)
        hin[1] = in_copy(1, 1)
        for s in range(NSUB):
            p = s % NBUF
            hin[p].wait()
            if s + 2 < NSUB:
                hin[(s + 2) % NBUF] = in_copy(s + 2, (s + 2) % NBUF)
            compute(bufs[p])

        @pl.when(is_tail)
        def _():
            pltpu.sync_copy(x_hbm.at[pl.ds(TAIL_BASE, TAILV * 16)], tailbuf)
            def f(i, c):
                one_vec(tailbuf, i * 16)
                return c
            lax.fori_loop(0, TAILV, f, 0)

    def fold(h, clear=True):
        def f(k, c):
            vs = [hist16[pl.ds(l * LSTRIDE + k * 16, 16)] for l in range(16)]
            if clear:
                for l in range(16):
                    hist16[pl.ds(l * LSTRIDE + k * 16, 16)] = zeros16
            while len(vs) > 1:
                vs = [a + b for a, b in zip(vs[::2], vs[1::2])]
            histf[pl.ds(k * 16, 16)] = vs[0]
            return c
        lax.fori_loop(0, h // 16, f, 0)

    def merge_and_cut(h, need, make_ctrl):
        r = h // NT
        pltpu.sync_copy(histf.at[pl.ds(0, h)], hist_sh.at[sid, pl.ds(0, h)])
        plsc.subcore_barrier()
        rbase = sid * r

        if r % 128 == 0:
            pltpu.sync_copy(hist_sh.at[pl.ds(0, NT), pl.ds(rbase, r)],
                            colbuf.at[pl.ds(0, NT), pl.ds(0, r)])
        else:
            for j in range(NT):
                pltpu.sync_copy(hist_sh.at[j, pl.ds(rbase, r)],
                                colbuf.at[j, pl.ds(0, r)])

        def af(i, c):
            vs = [colbuf[j, pl.ds(i * 16, 16)] for j in range(NT)]
            while len(vs) > 1:
                vs = [a + b for a, b in zip(vs[::2], vs[1::2])]
            rh[pl.ds(i * 16, 16)] = vs[0]
            return c
        lax.fori_loop(0, r // 16, af, 0)

        def tf(i, acc):
            return acc + rh[pl.ds(i * 16, 16)]
        total = jnp.sum(lax.fori_loop(0, r // 16, tf, zeros16))

        tmp[...] = jnp.broadcast_to(total, (16,)).astype(jnp.int32)
        pltpu.sync_copy(tmp, sum_sh.at[sid])
        plsc.subcore_barrier()
        pltpu.sync_copy(sum_sh, tot2d)
        totv = zeros16
        for j in range(NT):
            totv = totv + jnp.where(lanes == j, tot2d[j], 0)
        above = jnp.sum(jnp.where(lanes > sid, totv, 0))
        owner = (above < need) & (need <= above + total)

        @pl.when(owner)
        def _():
            a = above
            cutb = jnp.int32(0)
            acut = jnp.int32(0)
            for kk in range(r // 16 - 1, -1, -1):
                v = rh[pl.ds(kk * 16, 16)]
                rv = jnp.flip(v)
                cs = jnp.cumsum(rv)
                csx = cs - rv
                hit = ((a + csx) < need) & ((a + cs) >= need)
                bid = rbase + kk * 16 + 15 - lanes
                cutb = cutb + jnp.sum(jnp.where(hit, bid, 0))
                acut = acut + jnp.sum(jnp.where(hit, a + csx, 0))
                a = a + jnp.sum(v)
            tmp[...] = make_ctrl(cutb, need - acut)
            pltpu.sync_copy(tmp, ctrl_sh)
        plsc.subcore_barrier()
        pltpu.sync_copy(ctrl_sh, tmp2)
        cv = tmp2[...]

        def lane_of(k):
            return jnp.sum(jnp.where(lanes == k, cv, 0))
        return lane_of

    clear_hist(HIST)
    off0 = laneoff + jnp.int32(2048)
    scan_round(lambda ks: ([(k >> 20) + off0 for k in ks], None))
    fold(HIST)
    lane_of = merge_and_cut(
        HIST, jnp.int32(K),
        lambda cutb, neednext: (jnp.where(lanes == 0, cutb, 0)
                                + jnp.where(lanes == 3, neednext, 0)))
    b1 = lane_of(0)
    need2 = lane_of(3)


    b1m = b1 - jnp.int32(2048)

    def f2(ks):
        es = [k >> 20 for k in ks]
        ms = [e == b1m for e in es]
        ts = [k >> 8 for k in ks]
        ts = [t & jnp.int32(0xFFF) for t in ts]
        return [t + laneoff for t in ts], ms
    scan_round(f2)
    fold(HIST)
    lane_of = merge_and_cut(
        HIST, need2,
        lambda cutb, neednext: (jnp.where(lanes == 0, b1, 0)
                                + jnp.where(lanes == 1, cutb, 0)
                                + jnp.where(lanes == 3, neednext, 0)))
    c2 = lane_of(1)
    need3 = lane_of(3)

    pref24 = ((b1 - jnp.int32(2048)) << 12) + c2

    def f3(ks):
        ts = [k >> 8 for k in ks]
        ms = [t == pref24 for t in ts]
        ls = [k & jnp.int32(0xFF) for k in ks]
        return [l + laneoff for l in ls], ms
    scan_round(f3)
    fold(H3, clear=False)
    lane_of = merge_and_cut(
        H3, need3,
        lambda cutb, neednext: (jnp.where(lanes == 0, b1, 0)
                                + jnp.where(lanes == 1, c2, 0)
                                + jnp.where(lanes == 2, cutb, 0)
                                + jnp.where(lanes == 3, neednext, 0)))
    c3 = lane_of(2)
    ebudget = lane_of(3)
    cut = (pref24 << 8) + c3

    hv = histf[pl.ds((c3 >> 4) * 16, 16)]
    c_t = jnp.sum(jnp.where(lanes == (c3 & jnp.int32(15)), hv, 0))
    tmp[...] = jnp.broadcast_to(c_t, (16,)).astype(jnp.int32)
    pltpu.sync_copy(tmp, sum_sh.at[sid])
    plsc.subcore_barrier()
    pltpu.sync_copy(sum_sh, tot2d)
    totv = zeros16
    for j in range(NT):
        totv = totv + jnp.where(lanes == j, tot2d[j], 0)
    prefix = jnp.sum(jnp.where(lanes < sid, totv, 0))
    take = jnp.clip(ebudget - prefix, 0, c_t)

    simple = (take == 0) | (take == c_t)

    @pl.when(simple)
    def _():
        cutx = jnp.where(take == c_t, cut - 1, cut)

        def mgroup(ref, offs):
            vs = [ref[pl.ds(o, 16)] for o in offs]
            bs = [plsc.bitcast(v, jnp.int32) for v in vs]
            sg = [b >> 31 for b in bs]
            sg = [s & jnp.int32(0x7FFFFFFF) for s in sg]
            ks = [b ^ s for b, s in zip(bs, sg)]
            rs = [jnp.where(k > cutx, jnp.float32(1.0), jnp.float32(0.0))
                  for k in ks]
            for o, r in zip(offs, rs):
                ref[pl.ds(o, 16)] = r

        def mvec(ref, off):
            mgroup(ref, [off])

        def compute(ref):
            def f(i, c):
                for g in range(UNR // GRP):
                    b = (i * UNR + g * GRP) * 16
                    mgroup(ref, [b + u * 16 for u in range(GRP)])
                return c
            lax.fori_loop(0, VECS // UNR, f, 0)

        hin = [None] * NBUF
        hout = [None] * NBUF
        hin[0] = in_copy(0, 0)
        hin[1] = in_copy(1, 1)
        for s in range(NSUB):
            p = s % NBUF
            hin[p].wait()
            if s + 2 < NSUB:
                q = (s + 2) % NBUF
                if s - 2 >= 0:
                    hout[q].wait()
                hin[q] = in_copy(s + 2, q)
            compute(bufs[p])
            hout[p] = pltpu.async_copy(
                bufs[p], o_hbm.at[pl.ds(base + s * SUB, SUB)], outsems[p])
        for s in range(max(NSUB - NBUF, 0), NSUB):
            hout[s % NBUF].wait()

        @pl.when(is_tail)
        def _():
            pltpu.sync_copy(x_hbm.at[pl.ds(TAIL_BASE, TAILV * 16)], tailbuf)
            def f(i, c):
                mvec(tailbuf, i * 16)
                return c
            lax.fori_loop(0, TAILV, f, 0)
            pltpu.sync_copy(tailbuf, o_hbm.at[pl.ds(TAIL_BASE, TAILV * 16)])

    @pl.when(jnp.logical_not(simple))
    def _():
        def mvec(ref, off, cnt):
            v = ref[pl.ds(off, 16)]
            k = _key(v)
            eqi = (k == cut).astype(jnp.int32)
            cs = jnp.cumsum(eqi)
            sel = (eqi > 0) & ((cnt + cs - eqi) < take)
            ref[pl.ds(off, 16)] = jnp.where(
                (k > cut) | sel, jnp.float32(1.0), jnp.float32(0.0))
            return cnt + jnp.sum(eqi)

        def msub(s, cnt):
            pltpu.sync_copy(x_hbm.at[pl.ds(base + s * SUB, SUB)], bufs[0])
            cnt = lax.fori_loop(
                0, VECS, lambda i, c: mvec(bufs[0], i * 16, c), cnt)
            pltpu.sync_copy(bufs[0], o_hbm.at[pl.ds(base + s * SUB, SUB)])
            return cnt
        cnt = lax.fori_loop(0, NSUB, msub, jnp.int32(0))

        @pl.when(is_tail)
        def _():
            pltpu.sync_copy(x_hbm.at[pl.ds(TAIL_BASE, TAILV * 16)], tailbuf)
            lax.fori_loop(
                0, TAILV, lambda i, c: mvec(tailbuf, i * 16, c), cnt)
            pltpu.sync_copy(tailbuf, o_hbm.at[pl.ds(TAIL_BASE, TAILV * 16)])


@jax.jit
def kernel(x):
    mesh = plsc.VectorSubcoreMesh(
        core_axis_name="c", subcore_axis_name="s", num_cores=1,
        num_subcores=NT)
    return pl.kernel(
        _body,
        out_type=jax.ShapeDtypeStruct((N,), jnp.float32),
        mesh=mesh,
        compiler_params=pltpu.CompilerParams(needs_layout_passes=False),
        scratch_types=[
            pltpu.VMEM((SUB,), jnp.float32),
            pltpu.VMEM((SUB,), jnp.float32),
            pltpu.VMEM((SUB,), jnp.float32),
            pltpu.VMEM((SUB,), jnp.float32),
            pltpu.VMEM((TAILV * 16,), jnp.float32),
            pltpu.VMEM((HB,), jnp.int32),
            pltpu.VMEM((HIST,), jnp.int32),
            pltpu.VMEM((HIST // NT,), jnp.int32),
            pltpu.VMEM((NT, HIST // NT), jnp.int32),
            pltpu.VMEM((16,), jnp.int32),
            pltpu.VMEM((16,), jnp.int32),
            pltpu.VMEM((NT, 16), jnp.int32),
            pltpu.VMEM_SHARED((NT, HIST), jnp.int32),
            pltpu.VMEM_SHARED((NT, 16), jnp.int32),
            pltpu.VMEM_SHARED((16,), jnp.int32),
            pltpu.SemaphoreType.DMA,
            pltpu.SemaphoreType.DMA,
            pltpu.SemaphoreType.DMA,
            pltpu.SemaphoreType.DMA,
            pltpu.SemaphoreType.DMA,
            pltpu.SemaphoreType.DMA,
            pltpu.SemaphoreType.DMA,
            pltpu.SemaphoreType.DMA,
        ],
    )(x)


# speedup vs baseline: 1.0199x; 1.0199x over previous
import functools

import jax
import jax.numpy as jnp
from jax import lax
from jax.experimental import pallas as pl
from jax.experimental.pallas import tpu as pltpu
from jax.experimental.pallas import tpu_sc as plsc

N = 1_000_000
K = 50
NT = 16
PER = 62_496
SUB = 10_416
NSUB = PER // SUB
VECS = SUB // 16
UNR = 21
GRP = 21
NBUF = 4
TAIL_BASE = NT * PER
TAILV = (N - TAIL_BASE) // 16
HIST = 4096
H3 = 256
LSTRIDE = HIST + 8
HB = LSTRIDE * 16


def _key(v):
    b = plsc.bitcast(v, jnp.int32)
    return b ^ ((b >> 31) & jnp.int32(0x7FFFFFFF))


def _body(x_hbm, o_hbm, b0, b1_, b2, b3, tailbuf, hist16, histf, rh, colbuf,
          tmp, tmp2, tot2d, hist_sh, sum_sh, ctrl_sh,
          is0, is1, is2, is3, os0, os1, os2, os3):
    bufs = [b0, b1_, b2, b3]
    insems = [is0, is1, is2, is3]
    outsems = [os0, os1, os2, os3]
    sid = lax.axis_index("s")
    base = sid * PER
    lanes = lax.iota(jnp.int32, 16)
    laneoff = lanes * LSTRIDE
    ones = jnp.ones((16,), jnp.int32)
    zeros16 = jnp.zeros((16,), jnp.int32)
    is_tail = sid == NT - 1

    def clear_hist(h):
        def f(i, c):
            for l in range(16):
                hist16[pl.ds(l * LSTRIDE + i * 16, 16)] = zeros16
            return c
        lax.fori_loop(0, h // 16, f, 0)

    def in_copy(s, p):
        return pltpu.async_copy(
            x_hbm.at[pl.ds(base + s * SUB, SUB)], bufs[p], insems[p])

    def scan_round(bucket_fn):
        def group(ref, offs):
            vs = [ref[pl.ds(o, 16)] for o in offs]
            bs = [plsc.bitcast(v, jnp.int32) for v in vs]
            sg = [b >> 31 for b in bs]
            sg = [s & jnp.int32(0x7FFFFFFF) for s in sg]
            ks = [b ^ s for b, s in zip(bs, sg)]
            idxs, ms = bucket_fn(ks)
            if ms is None:
                for idx in idxs:
                    plsc.addupdate_scatter(hist16, [idx], ones)
            else:
                for idx, m in zip(idxs, ms):
                    plsc.addupdate_scatter(hist16, [idx], ones, mask=m)

        def one_vec(ref, off):
            group(ref, [off])

        def compute(ref):
            def f(i, c):
                for g in range(UNR // GRP):
                    b = (i * UNR + g * GRP) * 16
                    group(ref, [b + u * 16 for u in range(GRP)])
                return c
            lax.fori_loop(0, VECS // UNR, f, 0)

        hin = [None] * NBUF
        hin[0] = in_copy(0, 0)
        hin[1] = in_copy(1, 1)
        for s in range(NSUB):
            p = s % NBUF
            hin[p].wait()
            if s + 2 < NSUB:
                hin[(s + 2) % NBUF] = in_copy(s + 2, (s + 2) % NBUF)
            compute(bufs[p])

        @pl.when(is_tail)
        def _():
            pltpu.sync_copy(x_hbm.at[pl.ds(TAIL_BASE, TAILV * 16)], tailbuf)
            def f(i, c):
                one_vec(tailbuf, i * 16)
                return c
            lax.fori_loop(0, TAILV, f, 0)

    def fold(h, clear=True):
        def f(k, c):
            vs = [hist16[pl.ds(l * LSTRIDE + k * 16, 16)] for l in range(16)]
            if clear:
                for l in range(16):
                    hist16[pl.ds(l * LSTRIDE + k * 16, 16)] = zeros16
            while len(vs) > 1:
                vs = [a + b for a, b in zip(vs[::2], vs[1::2])]
            histf[pl.ds(k * 16, 16)] = vs[0]
            return c
        lax.fori_loop(0, h // 16, f, 0)

    def merge_and_cut(h, need, make_ctrl):
        r = h // NT
        pltpu.sync_copy(histf.at[pl.ds(0, h)], hist_sh.at[sid, pl.ds(0, h)])
        plsc.subcore_barrier()
        rbase = sid * r

        if r % 128 == 0:
            pltpu.sync_copy(hist_sh.at[pl.ds(0, NT), pl.ds(rbase, r)],
                            colbuf.at[pl.ds(0, NT), pl.ds(0, r)])
        else:
            for j in range(NT):
                pltpu.sync_copy(hist_sh.at[j, pl.ds(rbase, r)],
                                colbuf.at[j, pl.ds(0, r)])

        def af(i, c):
            vs = [colbuf[j, pl.ds(i * 16, 16)] for j in range(NT)]
            while len(vs) > 1:
                vs = [a + b for a, b in zip(vs[::2], vs[1::2])]
            rh[pl.ds(i * 16, 16)] = vs[0]
            return c
        lax.fori_loop(0, r // 16, af, 0)

        def tf(i, acc):
            return acc + rh[pl.ds(i * 16, 16)]
        total = jnp.sum(lax.fori_loop(0, r // 16, tf, zeros16))

        tmp[...] = jnp.broadcast_to(total, (16,)).astype(jnp.int32)
        pltpu.sync_copy(tmp, sum_sh.at[sid])
        plsc.subcore_barrier()
        pltpu.sync_copy(sum_sh, tot2d)
        totv = zeros16
        for j in range(NT):
            totv = totv + jnp.where(lanes == j, tot2d[j], 0)
        above = jnp.sum(jnp.where(lanes > sid, totv, 0))
        owner = (above < need) & (need <= above + total)

        @pl.when(owner)
        def _():
            a = above
            cutb = jnp.int32(0)
            acut = jnp.int32(0)
            for kk in range(r // 16 - 1, -1, -1):
                v = rh[pl.ds(kk * 16, 16)]
                rv = jnp.flip(v)
                cs = jnp.cumsum(rv)
                csx = cs - rv
                hit = ((a + csx) < need) & ((a + cs) >= need)
                bid = rbase + kk * 16 + 15 - lanes
                cutb = cutb + jnp.sum(jnp.where(hit, bid, 0))
                acut = acut + jnp.sum(jnp.where(hit, a + csx, 0))
                a = a + jnp.sum(v)
            tmp[...] = make_ctrl(cutb, need - acut)
            pltpu.sync_copy(tmp, ctrl_sh)
        plsc.subcore_barrier()
        pltpu.sync_copy(ctrl_sh, tmp2)
        cv = tmp2[...]

        def lane_of(k):
            return jnp.sum(jnp.where(lanes == k, cv, 0))
        return lane_of

    clear_hist(HIST)
    off0 = laneoff + jnp.int32(2048)
    scan_round(lambda ks: ([(k >> 20) + off0 for k in ks], None))
    fold(HIST)
    lane_of = merge_and_cut(
        HIST, jnp.int32(K),
        lambda cutb, neednext: (jnp.where(lanes == 0, cutb, 0)
                                + jnp.where(lanes == 3, neednext, 0)))
    b1 = lane_of(0)
    need2 = lane_of(3)


    b1m = b1 - jnp.int32(2048)

    def f2(ks):
        es = [k >> 20 for k in ks]
        ms = [e == b1m for e in es]
        ts = [k >> 8 for k in ks]
        ts = [t & jnp.int32(0xFFF) for t in ts]
        return [t + laneoff for t in ts], ms
    scan_round(f2)
    fold(HIST)
    lane_of = merge_and_cut(
        HIST, need2,
        lambda cutb, neednext: (jnp.where(lanes == 0, b1, 0)
                                + jnp.where(lanes == 1, cutb, 0)
                                + jnp.where(lanes == 3, neednext, 0)))
    c2 = lane_of(1)
    need3 = lane_of(3)

    pref24 = ((b1 - jnp.int32(2048)) << 12) + c2

    def f3(ks):
        ts = [k >> 8 for k in ks]
        ms = [t == pref24 for t in ts]
        ls = [k & jnp.int32(0xFF) for k in ks]
        return [l + laneoff for l in ls], ms
    scan_round(f3)
    fold(H3, clear=False)

    pltpu.sync_copy(histf.at[pl.ds(0, H3)], hist_sh.at[sid, pl.ds(0, H3)])
    plsc.subcore_barrier()
    pltpu.sync_copy(hist_sh.at[pl.ds(0, NT), pl.ds(0, H3)],
                    colbuf.at[pl.ds(0, NT), pl.ds(0, H3)])

    def af3(i, c):
        vs = [colbuf[j, pl.ds(i * 16, 16)] for j in range(NT)]
        while len(vs) > 1:
            vs = [a + b for a, b in zip(vs[::2], vs[1::2])]
        rh[pl.ds(i * 16, 16)] = vs[0]
        return c
    lax.fori_loop(0, H3 // 16, af3, 0)
    a3 = jnp.int32(0)
    cutb3 = jnp.int32(0)
    acut3 = jnp.int32(0)
    for kk in range(H3 // 16 - 1, -1, -1):
        v = rh[pl.ds(kk * 16, 16)]
        rv = jnp.flip(v)
        cs = jnp.cumsum(rv)
        csx = cs - rv
        hit = ((a3 + csx) < need3) & ((a3 + cs) >= need3)
        bid = kk * 16 + 15 - lanes
        cutb3 = cutb3 + jnp.sum(jnp.where(hit, bid, 0))
        acut3 = acut3 + jnp.sum(jnp.where(hit, a3 + csx, 0))
        a3 = a3 + jnp.sum(v)
    c3 = cutb3
    ebudget = need3 - acut3
    cut = (pref24 << 8) + c3

    totv = plsc.load_gather(colbuf, [lanes, jnp.broadcast_to(c3, (16,))])
    c_t = jnp.sum(jnp.where(lanes == sid, totv, 0))
    prefix = jnp.sum(jnp.where(lanes < sid, totv, 0))
    take = jnp.clip(ebudget - prefix, 0, c_t)

    simple = (take == 0) | (take == c_t)

    @pl.when(simple)
    def _():
        cutx = jnp.where(take == c_t, cut - 1, cut)

        def mgroup(ref, offs):
            vs = [ref[pl.ds(o, 16)] for o in offs]
            bs = [plsc.bitcast(v, jnp.int32) for v in vs]
            sg = [b >> 31 for b in bs]
            sg = [s & jnp.int32(0x7FFFFFFF) for s in sg]
            ks = [b ^ s for b, s in zip(bs, sg)]
            rs = [jnp.where(k > cutx, jnp.float32(1.0), jnp.float32(0.0))
                  for k in ks]
            for o, r in zip(offs, rs):
                ref[pl.ds(o, 16)] = r

        def mvec(ref, off):
            mgroup(ref, [off])

        def compute(ref):
            def f(i, c):
                for g in range(UNR // GRP):
                    b = (i * UNR + g * GRP) * 16
                    mgroup(ref, [b + u * 16 for u in range(GRP)])
                return c
            lax.fori_loop(0, VECS // UNR, f, 0)

        hin = [None] * NBUF
        hout = [None] * NBUF
        hin[0] = in_copy(0, 0)
        hin[1] = in_copy(1, 1)
        for s in range(NSUB):
            p = s % NBUF
            hin[p].wait()
            if s + 2 < NSUB:
                q = (s + 2) % NBUF
                if s - 2 >= 0:
                    hout[q].wait()
                hin[q] = in_copy(s + 2, q)
            compute(bufs[p])
            hout[p] = pltpu.async_copy(
                bufs[p], o_hbm.at[pl.ds(base + s * SUB, SUB)], outsems[p])
        for s in range(max(NSUB - NBUF, 0), NSUB):
            hout[s % NBUF].wait()

        @pl.when(is_tail)
        def _():
            pltpu.sync_copy(x_hbm.at[pl.ds(TAIL_BASE, TAILV * 16)], tailbuf)
            def f(i, c):
                mvec(tailbuf, i * 16)
                return c
            lax.fori_loop(0, TAILV, f, 0)
            pltpu.sync_copy(tailbuf, o_hbm.at[pl.ds(TAIL_BASE, TAILV * 16)])

    @pl.when(jnp.logical_not(simple))
    def _():
        def mvec(ref, off, cnt):
            v = ref[pl.ds(off, 16)]
            k = _key(v)
            eqi = (k == cut).astype(jnp.int32)
            cs = jnp.cumsum(eqi)
            sel = (eqi > 0) & ((cnt + cs - eqi) < take)
            ref[pl.ds(off, 16)] = jnp.where(
                (k > cut) | sel, jnp.float32(1.0), jnp.float32(0.0))
            return cnt + jnp.sum(eqi)

        def msub(s, cnt):
            pltpu.sync_copy(x_hbm.at[pl.ds(base + s * SUB, SUB)], bufs[0])
            cnt = lax.fori_loop(
                0, VECS, lambda i, c: mvec(bufs[0], i * 16, c), cnt)
            pltpu.sync_copy(bufs[0], o_hbm.at[pl.ds(base + s * SUB, SUB)])
            return cnt
        cnt = lax.fori_loop(0, NSUB, msub, jnp.int32(0))

        @pl.when(is_tail)
        def _():
            pltpu.sync_copy(x_hbm.at[pl.ds(TAIL_BASE, TAILV * 16)], tailbuf)
            lax.fori_loop(
                0, TAILV, lambda i, c: mvec(tailbuf, i * 16, c), cnt)
            pltpu.sync_copy(tailbuf, o_hbm.at[pl.ds(TAIL_BASE, TAILV * 16)])


@jax.jit
def kernel(x):
    mesh = plsc.VectorSubcoreMesh(
        core_axis_name="c", subcore_axis_name="s", num_cores=1,
        num_subcores=NT)
    return pl.kernel(
        _body,
        out_type=jax.ShapeDtypeStruct((N,), jnp.float32),
        mesh=mesh,
        compiler_params=pltpu.CompilerParams(needs_layout_passes=False),
        scratch_types=[
            pltpu.VMEM((SUB,), jnp.float32),
            pltpu.VMEM((SUB,), jnp.float32),
            pltpu.VMEM((SUB,), jnp.float32),
            pltpu.VMEM((SUB,), jnp.float32),
            pltpu.VMEM((TAILV * 16,), jnp.float32),
            pltpu.VMEM((HB,), jnp.int32),
            pltpu.VMEM((HIST,), jnp.int32),
            pltpu.VMEM((HIST // NT,), jnp.int32),
            pltpu.VMEM((NT, HIST // NT), jnp.int32),
            pltpu.VMEM((16,), jnp.int32),
            pltpu.VMEM((16,), jnp.int32),
            pltpu.VMEM((NT, 16), jnp.int32),
            pltpu.VMEM_SHARED((NT, HIST), jnp.int32),
            pltpu.VMEM_SHARED((NT, 16), jnp.int32),
            pltpu.VMEM_SHARED((16,), jnp.int32),
            pltpu.SemaphoreType.DMA,
            pltpu.SemaphoreType.DMA,
            pltpu.SemaphoreType.DMA,
            pltpu.SemaphoreType.DMA,
            pltpu.SemaphoreType.DMA,
            pltpu.SemaphoreType.DMA,
            pltpu.SemaphoreType.DMA,
            pltpu.SemaphoreType.DMA,
        ],
    )(x)
